# baseline (speedup 1.0000x reference)
import functools
import numpy as np
import jax
import jax.numpy as jnp
from jax import lax
from jax.experimental import pallas as pl
from jax.experimental.pallas import tpu as pltpu
from jax.experimental.pallas import tpu_sc as plsc

_FEATURE_DIMS = [119, 5, 12, 12, 10, 6, 6, 2, 2]
_OFFS = [0, 119, 124, 136, 148, 158, 164, 170, 172]
_NCAT = 9
_TOT = 174
_VPAD = 176
_EMB = 64
_NCOL = _NCAT + 32
_BLOCK = 33408
_CH = 800
_N = 100000
_NCHUNK = _N // _CH
_NW = 32
_KMAX = -(-_NCHUNK // _NW)


def _proj_body(e_ref, w1_ref, o_ref):
    o_ref[...] = jnp.dot(e_ref[...], w1_ref[...], preferred_element_type=jnp.float32)


def _sc_body(xt_hbm, a_hbm, out_hbm, cat_v, a_v, acc_t, sem):
    wid = lax.axis_index("s") * 2 + lax.axis_index("c")
    pltpu.sync_copy(a_hbm, a_v)

    def chunk(k, _):
        c = k * _NW + wid

        @pl.when(c < _NCHUNK)
        def _():
            base = c * _CH
            for i in range(_NCAT):
                pltpu.sync_copy(xt_hbm.at[pl.ds(i * _N + base, _CH)],
                                cat_v.at[pl.ds(i * _CH, _CH)])

            def group(g, _):
                r0 = g * 16
                bases = []
                for i in range(_NCAT):
                    v = cat_v[pl.ds(i * _CH + r0, 16)]
                    vi = v.astype(jnp.int32) + _OFFS[i]
                    bases.append(vi * _EMB)
                for d in range(_EMB):
                    acc = plsc.load_gather(a_v, [bases[0] + d])
                    for i in range(1, _NCAT):
                        acc = acc + plsc.load_gather(a_v, [bases[i] + d])
                    acc_t[pl.ds(d * _CH + r0, 16)] = acc
                return 0

            lax.fori_loop(0, _CH // 16, group, 0)
            cps = [
                pltpu.async_copy(acc_t.at[pl.ds(d * _CH, _CH)],
                                 out_hbm.at[pl.ds(d * _N + base, _CH)], sem)
                for d in range(_EMB)
            ]
            for cp in cps:
                cp.wait()
        return 0

    lax.fori_loop(0, _KMAX, chunk, 0)


def _fin_body(xt_ref, pt_ref, w2t_ref, b_ref, o_ref):
    xt = xt_ref[...]
    sig = jnp.dot(w2t_ref[...], xt[_NCAT:, :].astype(jnp.bfloat16),
                  preferred_element_type=jnp.float32)
    o_ref[...] = pt_ref[...] + sig + b_ref[...]


@jax.jit
def kernel(x, emb_0, emb_1, emb_2, emb_3, emb_4, emb_5, emb_6, emb_7, emb_8, W, b):
    n = x.shape[0]
    xt = x.T
    tables = [emb_0, emb_1, emb_2, emb_3, emb_4, emb_5, emb_6, emb_7, emb_8]
    e = jnp.concatenate(
        tables + [jnp.zeros((_VPAD - _TOT, _EMB), jnp.float32)], axis=0
    )
    w2t = W[_EMB:, :].T.astype(jnp.bfloat16)
    b2 = b.reshape(_EMB, 1)

    a = pl.pallas_call(
        _proj_body,
        out_shape=jax.ShapeDtypeStruct((_VPAD, _EMB), jnp.float32),
    )(e, W[:_EMB, :])
    a_flat = a.reshape(-1)

    mesh = plsc.VectorSubcoreMesh(core_axis_name="c", subcore_axis_name="s")
    sc = functools.partial(
        pl.kernel,
        out_type=jax.ShapeDtypeStruct((_EMB * n,), jnp.float32),
        mesh=mesh,
        scratch_types=[
            pltpu.VMEM((_NCAT * _CH,), jnp.float32),
            pltpu.VMEM((_VPAD * _EMB,), jnp.float32),
            pltpu.VMEM((_EMB * _CH,), jnp.float32),
            pltpu.SemaphoreType.DMA,
        ],
        compiler_params=pltpu.CompilerParams(needs_layout_passes=False),
    )(_sc_body)
    pt = sc(xt.reshape(-1), a_flat).reshape(_EMB, n)

    grid = (pl.cdiv(n, _BLOCK),)
    outt = pl.pallas_call(
        _fin_body,
        grid=grid,
        in_specs=[
            pl.BlockSpec((_NCOL, _BLOCK), lambda i: (0, i)),
            pl.BlockSpec((_EMB, _BLOCK), lambda i: (0, i)),
            pl.BlockSpec((_EMB, _NCOL - _NCAT), lambda i: (0, 0)),
            pl.BlockSpec((_EMB, 1), lambda i: (0, 0)),
        ],
        out_specs=pl.BlockSpec((_EMB, _BLOCK), lambda i: (0, i)),
        out_shape=jax.ShapeDtypeStruct((_EMB, n), jnp.float32),
        compiler_params=pltpu.CompilerParams(
            dimension_semantics=("arbitrary",),
        ),
    )(xt, pt, w2t, b2)
    return outt.T


# speedup vs baseline: 9.3680x; 9.3680x over previous
import numpy as np
import jax
import jax.numpy as jnp
from jax.experimental import pallas as pl
from jax.experimental.pallas import tpu as pltpu

_FEATURE_DIMS = [119, 5, 12, 12, 10, 6, 6, 2, 2]
_OFFS = [0, 119, 124, 136, 148, 158, 164, 170, 172]
_NCAT = 9
_TOT = 174
_VPAD = 176
_EMB = 64
_NCOL = _NCAT + 32
_BLOCK = 33408

_S = np.zeros((_VPAD, _NCAT), np.float32)
_T = np.full((_VPAD, 1), -1.0, np.float32)
for _i in range(_NCAT):
    _lo = _OFFS[_i]
    _hi = _lo + _FEATURE_DIMS[_i]
    _S[_lo:_hi, _i] = 1.0
    _T[_lo:_hi, 0] = np.arange(_hi - _lo, dtype=np.float32)


def _body(xt_ref, st_ref, tt_ref, et_ref, w1t_ref, w2t_ref, b_ref, o_ref):
    xt = xt_ref[...]
    cat = jnp.trunc(xt[:_NCAT, :]).astype(jnp.bfloat16)
    c = jnp.dot(st_ref[...], cat, preferred_element_type=jnp.float32)
    oh = (c == tt_ref[...]).astype(jnp.bfloat16)
    a = jnp.dot(w1t_ref[...], et_ref[...], preferred_element_type=jnp.float32)
    emb = jnp.dot(a.astype(jnp.bfloat16), oh, preferred_element_type=jnp.float32)
    sig = jnp.dot(w2t_ref[...], xt[_NCAT:, :].astype(jnp.bfloat16),
                  preferred_element_type=jnp.float32)
    o_ref[...] = emb + sig + b_ref[...]


@jax.jit
def kernel(x, emb_0, emb_1, emb_2, emb_3, emb_4, emb_5, emb_6, emb_7, emb_8, W, b):
    n = x.shape[0]
    xt = x.T
    tables = [emb_0, emb_1, emb_2, emb_3, emb_4, emb_5, emb_6, emb_7, emb_8]
    et = jnp.concatenate(
        tables + [jnp.zeros((_VPAD - _TOT, _EMB), jnp.float32)], axis=0
    ).T
    w1t = W[:_EMB, :].T
    w2t = W[_EMB:, :].T
    st = jnp.asarray(_S, dtype=jnp.bfloat16)
    tt = jnp.asarray(_T)
    w2t = w2t.astype(jnp.bfloat16)
    b2 = b.reshape(_EMB, 1)
    grid = (pl.cdiv(n, _BLOCK),)
    outt = pl.pallas_call(
        _body,
        grid=grid,
        in_specs=[
            pl.BlockSpec((_NCOL, _BLOCK), lambda i: (0, i)),
            pl.BlockSpec((_VPAD, _NCAT), lambda i: (0, 0)),
            pl.BlockSpec((_VPAD, 1), lambda i: (0, 0)),
            pl.BlockSpec((_EMB, _VPAD), lambda i: (0, 0)),
            pl.BlockSpec((_EMB, _EMB), lambda i: (0, 0)),
            pl.BlockSpec((_EMB, _NCOL - _NCAT), lambda i: (0, 0)),
            pl.BlockSpec((_EMB, 1), lambda i: (0, 0)),
        ],
        out_specs=pl.BlockSpec((_EMB, _BLOCK), lambda i: (0, i)),
        out_shape=jax.ShapeDtypeStruct((_EMB, n), jnp.float32),
        compiler_params=pltpu.CompilerParams(
            dimension_semantics=("arbitrary",),
        ),
    )(xt, st, tt, et, w1t, w2t, b2)
    return outt.T


# speedup vs baseline: 9.3839x; 1.0017x over previous
import numpy as np
import jax
import jax.numpy as jnp
from jax.experimental import pallas as pl
from jax.experimental.pallas import tpu as pltpu

_FEATURE_DIMS = [119, 5, 12, 12, 10, 6, 6, 2, 2]
_OFFS = [0, 119, 124, 136, 148, 158, 164, 170, 172]
_NCAT = 9
_TOT = 174
_VPAD = 176
_EMB = 64
_NCOL = _NCAT + 32
_BLOCK = 33408

_S = np.zeros((_VPAD, _NCAT), np.float32)
_T = np.full((_VPAD, 1), -1.0, np.float32)
for _i in range(_NCAT):
    _lo = _OFFS[_i]
    _hi = _lo + _FEATURE_DIMS[_i]
    _S[_lo:_hi, _i] = 1.0
    _T[_lo:_hi, 0] = np.arange(_hi - _lo, dtype=np.float32)


def _body(xt_ref, st_ref, tt_ref, et_ref, w1t_ref, w2t_ref, b_ref, o_ref):
    xt = xt_ref[...]
    cat = jnp.trunc(xt[:_NCAT, :]).astype(jnp.bfloat16)
    c = jnp.dot(st_ref[...], cat, preferred_element_type=jnp.float32)
    oh = (c == tt_ref[...]).astype(jnp.bfloat16)
    a = jnp.dot(w1t_ref[...], et_ref[...], preferred_element_type=jnp.float32)
    emb = jnp.dot(a.astype(jnp.bfloat16), oh, preferred_element_type=jnp.float32)
    sig = jnp.dot(w2t_ref[...], xt[_NCAT:, :].astype(jnp.bfloat16),
                  preferred_element_type=jnp.float32)
    o_ref[...] = emb + sig + b_ref[...]


@jax.jit
def kernel(x, emb_0, emb_1, emb_2, emb_3, emb_4, emb_5, emb_6, emb_7, emb_8, W, b):
    n = x.shape[0]
    xt = x.T
    tables = [emb_0, emb_1, emb_2, emb_3, emb_4, emb_5, emb_6, emb_7, emb_8]
    et = jnp.concatenate(
        tables + [jnp.zeros((_VPAD - _TOT, _EMB), jnp.float32)], axis=0
    ).T
    w1t = W[:_EMB, :].T
    w2t = W[_EMB:, :].T
    st = jnp.asarray(_S, dtype=jnp.bfloat16)
    tt = jnp.asarray(_T)
    w2t = w2t.astype(jnp.bfloat16)
    b2 = b.reshape(_EMB, 1)
    grid = (pl.cdiv(n, _BLOCK),)
    outt = pl.pallas_call(
        _body,
        grid=grid,
        in_specs=[
            pl.BlockSpec((_NCOL, _BLOCK), lambda i: (0, i)),
            pl.BlockSpec((_VPAD, _NCAT), lambda i: (0, 0)),
            pl.BlockSpec((_VPAD, 1), lambda i: (0, 0)),
            pl.BlockSpec((_EMB, _VPAD), lambda i: (0, 0)),
            pl.BlockSpec((_EMB, _EMB), lambda i: (0, 0)),
            pl.BlockSpec((_EMB, _NCOL - _NCAT), lambda i: (0, 0)),
            pl.BlockSpec((_EMB, 1), lambda i: (0, 0)),
        ],
        out_specs=pl.BlockSpec((_EMB, _BLOCK), lambda i: (0, i)),
        out_shape=jax.ShapeDtypeStruct((_EMB, n), jnp.float32),
        compiler_params=pltpu.CompilerParams(
            dimension_semantics=("parallel",),
        ),
    )(xt, st, tt, et, w1t, w2t, b2)
    return outt.T
